# baseline (speedup 1.0000x reference)
import functools

import jax
import jax.numpy as jnp
from jax.experimental import pallas as pl
from jax.experimental.pallas import tpu as pltpu

SUB = 8
WA = 8192
WB = 2048


def _rotl(x, d):
    return (x << jnp.uint32(d)) | (x >> jnp.uint32(32 - d))


def _threefry_bits(j):
    ks0 = jnp.uint32(0)
    ks1 = jnp.uint32(1)
    ks2 = jnp.uint32(0x1BD11BDA) ^ ks0 ^ ks1
    ks = (ks0, ks1, ks2)
    rotations = ((13, 15, 26, 6), (17, 29, 16, 24))
    x0 = jnp.zeros_like(j) + ks0
    x1 = j + ks1
    for i in range(5):
        for r in rotations[i % 2]:
            x0 = x0 + x1
            x1 = _rotl(x1, r)
            x1 = x1 ^ x0
        x0 = x0 + ks[(i + 1) % 3]
        x1 = x1 + ks[(i + 2) % 3] + jnp.uint32(i + 1)
    return x0 ^ x1


def _stats_body(temps_ref, logits_ref, m_ref, g_ref, s_ref,
                m_acc, g_acc, s_acc, *, C, nblk):
    r = pl.program_id(0)
    c = pl.program_id(1)

    @pl.when(c == 0)
    def _init():
        m_acc[0] = -jnp.inf
        g_acc[0] = jnp.int32(SUB * C)
        s_acc[0] = jnp.float32(0.0)

    t_raw = temps_ref[r]
    t = jnp.where(t_raw < 1e-5, jnp.float32(1.0), t_raw)
    rt = jnp.float32(1.0) / t

    x = logits_ref[...]
    scaled = x * rt
    sub = jax.lax.broadcasted_iota(jnp.int32, x.shape, 1)
    lane = jax.lax.broadcasted_iota(jnp.int32, x.shape, 2)
    col = c * WA + lane
    valid = col < C
    flat = sub * C + col
    big = jnp.int32(SUB * C)

    neginf = jnp.float32(-jnp.inf)
    sc = jnp.where(valid, scaled, neginf)
    chm = jnp.max(sc)
    chidx = jnp.min(jnp.where(sc == chm, flat, big))

    m_old = m_acc[0]
    m_new = jnp.maximum(m_old, chm)
    bsum = jnp.sum(jnp.where(valid, jnp.exp(scaled - m_new), jnp.float32(0.0)))
    s_acc[0] = s_acc[0] * jnp.exp(m_old - m_new) + bsum
    g_acc[0] = jnp.where(chm > m_old, chidx, g_acc[0])
    m_acc[0] = m_new

    @pl.when(c == nblk - 1)
    def _emit():
        m_ref[...] = jnp.reshape(m_acc[0], (1, 1, 1))
        g_ref[...] = jnp.reshape(g_acc[0], (1, 1, 1))
        s_ref[...] = jnp.reshape(s_acc[0], (1, 1, 1))


def _sample_body(temps_ref, m_ref, s_ref, g_ref, logits_ref, probs_ref, tok_ref,
                 best_acc, bidx_acc, nan_acc, *, V, C, nblk):
    r = pl.program_id(0)
    c = pl.program_id(1)

    @pl.when(c == 0)
    def _init():
        best_acc[0] = -jnp.inf
        bidx_acc[0] = jnp.int32(0)
        nan_acc[0] = jnp.int32(V)

    t_raw = temps_ref[r]
    t = jnp.where(t_raw < 1e-5, jnp.float32(1.0), t_raw)
    rt = jnp.float32(1.0) / t
    m = m_ref[r]
    rs = jnp.float32(1.0) / s_ref[r]

    x = logits_ref[...]
    e = jnp.exp(x * rt - m)
    probs = e * rs
    probs_ref[...] = probs

    sub = jax.lax.broadcasted_iota(jnp.int32, x.shape, 1)
    lane = jax.lax.broadcasted_iota(jnp.int32, x.shape, 2)
    col = c * WB + lane
    valid = col < C
    flat = sub * C + col
    big = jnp.int32(V)

    j = (r * V + flat).astype(jnp.uint32)
    bits = _threefry_bits(j)
    uf = jax.lax.bitcast_convert_type(
        (bits >> jnp.uint32(9)) | jnp.uint32(0x3F800000), jnp.float32
    ) - jnp.float32(1.0)
    q = -jnp.log1p(-uf)
    ratio = probs / q

    ok = valid & (ratio == ratio)
    neginf = jnp.float32(-jnp.inf)
    r2 = jnp.where(ok, ratio, neginf)
    cmx = jnp.max(r2)
    cidx = jnp.min(jnp.where(r2 == cmx, flat, big))
    cnan = jnp.min(jnp.where(valid & (ratio != ratio), flat, big))

    best_old = best_acc[0]
    bidx_acc[0] = jnp.where(cmx > best_old, cidx, bidx_acc[0])
    best_acc[0] = jnp.maximum(best_old, cmx)
    nan_acc[0] = jnp.minimum(nan_acc[0], cnan)

    @pl.when(c == nblk - 1)
    def _emit():
        sampled = jnp.where(nan_acc[0] < big, nan_acc[0], bidx_acc[0])
        tok = jnp.where(t_raw < 1e-5, g_ref[r], sampled)
        tok_ref[...] = jnp.reshape(tok, (1, 1, 1))


def kernel(logits, temperatures):
    B, V = logits.shape
    C = V // SUB
    x3 = logits.reshape(B, SUB, C)

    nblk_a = pl.cdiv(C, WA)
    m3, g3, s3 = pl.pallas_call(
        functools.partial(_stats_body, C=C, nblk=nblk_a),
        grid=(B, nblk_a),
        in_specs=[
            pl.BlockSpec(memory_space=pltpu.SMEM),
            pl.BlockSpec((1, SUB, WA), lambda r, c: (r, 0, c)),
        ],
        out_specs=[
            pl.BlockSpec((1, 1, 1), lambda r, c: (r, 0, 0)),
            pl.BlockSpec((1, 1, 1), lambda r, c: (r, 0, 0)),
            pl.BlockSpec((1, 1, 1), lambda r, c: (r, 0, 0)),
        ],
        out_shape=[
            jax.ShapeDtypeStruct((B, 1, 1), jnp.float32),
            jax.ShapeDtypeStruct((B, 1, 1), jnp.int32),
            jax.ShapeDtypeStruct((B, 1, 1), jnp.float32),
        ],
        scratch_shapes=[
            pltpu.SMEM((1,), jnp.float32),
            pltpu.SMEM((1,), jnp.int32),
            pltpu.SMEM((1,), jnp.float32),
        ],
    )(temperatures, x3)

    nblk_b = pl.cdiv(C, WB)
    probs3, tok3 = pl.pallas_call(
        functools.partial(_sample_body, V=V, C=C, nblk=nblk_b),
        grid=(B, nblk_b),
        in_specs=[
            pl.BlockSpec(memory_space=pltpu.SMEM),
            pl.BlockSpec(memory_space=pltpu.SMEM),
            pl.BlockSpec(memory_space=pltpu.SMEM),
            pl.BlockSpec(memory_space=pltpu.SMEM),
            pl.BlockSpec((1, SUB, WB), lambda r, c: (r, 0, c)),
        ],
        out_specs=[
            pl.BlockSpec((1, SUB, WB), lambda r, c: (r, 0, c)),
            pl.BlockSpec((1, 1, 1), lambda r, c: (r, 0, 0)),
        ],
        out_shape=[
            jax.ShapeDtypeStruct((B, SUB, C), jnp.float32),
            jax.ShapeDtypeStruct((B, 1, 1), jnp.int32),
        ],
        scratch_shapes=[
            pltpu.SMEM((1,), jnp.float32),
            pltpu.SMEM((1,), jnp.int32),
            pltpu.SMEM((1,), jnp.int32),
        ],
    )(temperatures, m3.reshape(B), s3.reshape(B), g3.reshape(B), x3)

    return tok3.reshape(B), probs3.reshape(B, V)
